# baseline (speedup 1.0000x reference)
import jax
import jax.numpy as jnp
from jax import lax
from jax.experimental import pallas as pl
from jax.experimental.pallas import tpu as pltpu
from jax.experimental.pallas import tpu_sc as plsc

NC = 2
NS = 16
L = 16
NW = NC * NS

B = 16384
D = 128
CH = 128
BPW = B // NW
NCHUNK = BPW // CH
GROUPS = CH // L
NVEC = D // L


def _body(user_h, item_h, uf_h, if_h, ub_h, ib_h, out_h,
          uidx, iidx, urows0, urows1, irows0, irows1, ubias, ibias,
          accm, outv, sem0, sem1):
    cid = lax.axis_index("c")
    sid = lax.axis_index("s")
    wid = sid * NC + cid
    base = wid * BPW

    for ch in range(NCHUNK):
        pltpu.sync_copy(user_h.at[pl.ds(base + ch * CH, CH)], uidx.at[ch])
        pltpu.sync_copy(item_h.at[pl.ds(base + ch * CH, CH)], iidx.at[ch])

    urows = (urows0, urows1)
    irows = (irows0, irows1)
    sems = (sem0, sem1)

    def fire(ch):
        b = ch % 2
        s = sems[b]
        return (
            pltpu.async_copy(uf_h.at[uidx.at[ch]], urows[b], s),
            pltpu.async_copy(if_h.at[iidx.at[ch]], irows[b], s),
            pltpu.async_copy(ub_h.at[uidx.at[ch]], ubias.at[ch], s),
            pltpu.async_copy(ib_h.at[iidx.at[ch]], ibias.at[ch], s),
        )

    iota = lax.iota(jnp.int32, L)

    def compute(ch):
        b = ch % 2
        u = urows[b]
        v = irows[b]

        def group_body(g, _):
            rowbase = g * L
            for r in range(L):
                row = rowbase + r
                acc = u[row, pl.ds(0, L)] * v[row, pl.ds(0, L)]
                for j in range(1, NVEC):
                    acc = acc + u[row, pl.ds(j * L, L)] * v[row, pl.ds(j * L, L)]
                plsc.store_scatter(accm, [iota, jnp.full((L,), r, jnp.int32)], acc)
            tot = accm[0, :]
            for j in range(1, L):
                tot = tot + accm[j, :]
            tot = tot + ubias[ch, pl.ds(rowbase, L)] + ibias[ch, pl.ds(rowbase, L)]
            outv[pl.ds(ch * CH + rowbase, L)] = tot
            return 0

        lax.fori_loop(0, GROUPS, group_body, 0)

    pending = {0: fire(0)}
    for ch in range(NCHUNK):
        if ch + 1 < NCHUNK:
            pending[ch + 1] = fire(ch + 1)
        for d in pending.pop(ch):
            d.wait()
        compute(ch)

    pltpu.sync_copy(outv, out_h.at[pl.ds(base, BPW)])


@jax.jit
def _run(user, item, user_factors, item_factors, ub, ib):
    mesh = plsc.VectorSubcoreMesh(core_axis_name="c", subcore_axis_name="s")
    f = pl.kernel(
        _body,
        out_type=jax.ShapeDtypeStruct((B,), jnp.float32),
        mesh=mesh,
        compiler_params=pltpu.CompilerParams(needs_layout_passes=False),
        scratch_types=[
            pltpu.VMEM((NCHUNK, CH), jnp.int32),
            pltpu.VMEM((NCHUNK, CH), jnp.int32),
            pltpu.VMEM((CH, D), jnp.float32),
            pltpu.VMEM((CH, D), jnp.float32),
            pltpu.VMEM((CH, D), jnp.float32),
            pltpu.VMEM((CH, D), jnp.float32),
            pltpu.VMEM((NCHUNK, CH), jnp.float32),
            pltpu.VMEM((NCHUNK, CH), jnp.float32),
            pltpu.VMEM((L, L), jnp.float32),
            pltpu.VMEM((BPW,), jnp.float32),
            pltpu.SemaphoreType.DMA,
            pltpu.SemaphoreType.DMA,
        ],
    )
    return f(user, item, user_factors, item_factors, ub, ib)


def kernel(user, item, user_factors, item_factors, user_biases, item_biases):
    ub = user_biases.reshape(-1)
    ib = item_biases.reshape(-1)
    return _run(user, item, user_factors, item_factors, ub, ib)
